# baseline (speedup 1.0000x reference)
import jax
import jax.numpy as jnp
from jax.experimental import pallas as pl
from jax.experimental.pallas import tpu as pltpu

D_EMBED = 2048
N_EXP = 16
CAP = 512
N_BATCH = 2
S_SEQ = 4096

ROW_TILE = 512
N_TILES = S_SEQ // ROW_TILE


def _fused_body(x_ref, wt_ref, logits_ref, probs_ref, mask_ref, pbits_ref):
    t = pl.program_id(1)

    l = jnp.dot(x_ref[0], wt_ref[...], preferred_element_type=jnp.float32)
    m = jnp.max(l, axis=-1, keepdims=True)
    e = jnp.exp(l - m)
    p = e / jnp.sum(e, axis=-1, keepdims=True)
    logits_ref[...] = l[None]
    probs_ref[...] = p[None]
    pbits_ref[pl.ds(t * ROW_TILE, ROW_TILE), :] = jax.lax.bitcast_convert_type(
        p, jnp.int32
    )

    @pl.when(t == N_TILES - 1)
    def _mask():
        bits = pbits_ref[...]

        lo0 = jnp.zeros((1, N_EXP), jnp.int32)
        hi0 = jnp.full((1, N_EXP), 0x3F800001, jnp.int32)

        def step(_, lohi):
            lo, hi = lohi
            mid = lo + (hi - lo) // 2
            cnt = jnp.sum((bits >= mid).astype(jnp.int32), axis=0, keepdims=True)
            ge = cnt >= CAP
            return (jnp.where(ge, mid, lo), jnp.where(ge, hi, mid))

        lo, _ = jax.lax.fori_loop(0, 31, step, (lo0, hi0))
        thr = lo

        gt = bits > thr
        n_gt = jnp.sum(gt.astype(jnp.int32), axis=0, keepdims=True)
        rem = CAP - n_gt

        eq = (bits == thr).astype(jnp.int32)
        pref = eq
        sh = 1
        while sh < S_SEQ:
            pref = pref + jnp.pad(pref, ((sh, 0), (0, 0)))[:S_SEQ, :]
            sh *= 2
        take_eq = (eq > 0) & (pref <= rem)

        sel = gt | take_eq
        any_sel = jnp.any(sel, axis=-1, keepdims=True)
        col = jax.lax.broadcasted_iota(jnp.int32, (1, N_EXP), 1)
        mask_ref[...] = jnp.where((col == 0) & any_sel, 1.0, 0.0)[None]


@jax.jit
def kernel(x, W):
    wt = W.T

    logits, probs, mask = pl.pallas_call(
        _fused_body,
        grid=(N_BATCH, N_TILES),
        in_specs=[
            pl.BlockSpec((1, ROW_TILE, D_EMBED), lambda b, t: (b, t, 0)),
            pl.BlockSpec((D_EMBED, N_EXP), lambda b, t: (0, 0)),
        ],
        out_specs=[
            pl.BlockSpec((1, ROW_TILE, N_EXP), lambda b, t: (b, t, 0)),
            pl.BlockSpec((1, ROW_TILE, N_EXP), lambda b, t: (b, t, 0)),
            pl.BlockSpec((1, S_SEQ, N_EXP), lambda b, t: (b, 0, 0)),
        ],
        out_shape=[
            jax.ShapeDtypeStruct((N_BATCH, S_SEQ, N_EXP), jnp.float32),
            jax.ShapeDtypeStruct((N_BATCH, S_SEQ, N_EXP), jnp.float32),
            jax.ShapeDtypeStruct((N_BATCH, S_SEQ, N_EXP), jnp.float32),
        ],
        scratch_shapes=[pltpu.VMEM((S_SEQ, N_EXP), jnp.int32)],
    )(x, wt)

    return (mask, probs, logits)
